# baseline (speedup 1.0000x reference)
import functools

import jax
import jax.numpy as jnp
from jax import lax
from jax.experimental import pallas as pl
from jax.experimental.pallas import tpu as pltpu
from jax.experimental.pallas import tpu_sc as plsc

N_NODES = 10000
D_FEAT = 128
D_EDGE = 16
HIDDEN = 128
N_MOLS = 400
N_TASKS = 8
DEPTH = 3

NC = 2
NS = 16
NW = NC * NS
CH = 128
SCW = 16
NBUF = 2
NACC = 10112
RPS = NACC // NS
SENT = N_NODES

def _mesh():
    return plsc.VectorSubcoreMesh(core_axis_name="c", subcore_axis_name="s")


def _seg_sum_gather(nsc0, nsc1):
    nsc_max = max(nsc0, nsc1)

    @functools.partial(
        pl.kernel,
        out_type=jax.ShapeDtypeStruct((NC, NACC, HIDDEN), jnp.float32),
        mesh=_mesh(),
        scratch_types=[
            pltpu.VMEM((2, SCW, CH), jnp.int32),
            pltpu.VMEM((2, SCW, CH), jnp.int32),
            pltpu.VMEM((NBUF, CH, HIDDEN), jnp.float32),
            pltpu.VMEM_SHARED((NACC, HIDDEN), jnp.float32),
            [pltpu.SemaphoreType.DMA] * NBUF,
            pltpu.SemaphoreType.DMA,
            pltpu.SemaphoreType.DMA,
        ],
    )
    def k(table0, table1, srci, dsti, zeros, out, src_v, dst_v, bufs, acc,
          sems, sem_si, sem_di):
        cid = lax.axis_index("c")
        sid = lax.axis_index("s")
        w = cid * NS + sid
        pltpu.sync_copy(zeros.at[pl.ds(sid * RPS, RPS)], acc.at[pl.ds(sid * RPS, RPS)])
        pltpu.async_copy(srci.at[w, 0], src_v.at[0], sem_si)
        pltpu.async_copy(dsti.at[w, 0], dst_v.at[0], sem_di)
        plsc.subcore_barrier()

        def go(table, ns):
            @pl.loop(0, ns)
            def _(t):
                rt = lax.rem(t, 2)
                pltpu.make_async_copy(srci.at[w, 0], src_v.at[0], sem_si).wait()
                pltpu.make_async_copy(dsti.at[w, 0], dst_v.at[0], sem_di).wait()
                pltpu.async_copy(srci.at[w, t + 1], src_v.at[1 - rt], sem_si)
                pltpu.async_copy(dsti.at[w, t + 1], dst_v.at[1 - rt], sem_di)
                for b in range(NBUF):
                    pltpu.async_copy(table.at[src_v.at[rt, b]], bufs.at[b], sems[b])

                @pl.loop(0, SCW - NBUF, step=NBUF)
                def _(j):
                    for b in range(NBUF):
                        pltpu.make_async_copy(
                            table.at[src_v.at[rt, 0]], bufs.at[b], sems[b]
                        ).wait()
                        pltpu.sync_copy(
                            bufs.at[b], acc.at[dst_v.at[rt, j + b]], add=True
                        )
                        pltpu.async_copy(
                            table.at[src_v.at[rt, j + b + NBUF]], bufs.at[b], sems[b]
                        )

                for b in range(NBUF):
                    pltpu.make_async_copy(
                        table.at[src_v.at[rt, 0]], bufs.at[b], sems[b]
                    ).wait()
                    pltpu.sync_copy(
                        bufs.at[b], acc.at[dst_v.at[rt, SCW - NBUF + b]], add=True
                    )

        @pl.when(cid == 0)
        def _():
            go(table0, nsc0)

        @pl.when(cid == 1)
        def _():
            go(table1, nsc1)

        pltpu.make_async_copy(srci.at[w, 0], src_v.at[0], sem_si).wait()
        pltpu.make_async_copy(dsti.at[w, 0], dst_v.at[0], sem_di).wait()
        plsc.subcore_barrier()
        pltpu.sync_copy(acc.at[pl.ds(sid * RPS, RPS)], out.at[cid].at[pl.ds(sid * RPS, RPS)])

    return k


def _tc_input(x, w_i):
    def body(x_ref, w_ref, o_ref):
        o_ref[...] = jnp.maximum(
            jnp.dot(x_ref[...], w_ref[...], preferred_element_type=jnp.float32), 0.0
        )

    return pl.pallas_call(
        body, out_shape=jax.ShapeDtypeStruct((N_NODES, HIDDEN), jnp.float32)
    )(x, w_i)


_EP_C = 8192


def _tc_edge_proj(ea, w_h2):
    n = ea.shape[0]
    grid = n // _EP_C

    def body(ea_ref, w_ref, z_ref):
        z_ref[...] = jnp.dot(ea_ref[...], w_ref[...], preferred_element_type=jnp.float32)

    return pl.pallas_call(
        body,
        grid=(grid,),
        in_specs=[
            pl.BlockSpec((_EP_C, D_EDGE), lambda k: (k, 0)),
            pl.BlockSpec((D_EDGE, HIDDEN), lambda k: (0, 0)),
        ],
        out_specs=pl.BlockSpec((_EP_C, HIDDEN), lambda k: (k, 0)),
        out_shape=jax.ShapeDtypeStruct((n, HIDDEN), jnp.float32),
    )(ea, w_h2)


def _tc_update(h0, s_p, be_p, w_h1):
    def body(h0_ref, s_ref, be_ref, w1_ref, h_ref):
        s = s_ref[0, :N_NODES, :] + s_ref[1, :N_NODES, :]
        be = be_ref[0, :N_NODES, :] + be_ref[1, :N_NODES, :]
        upd = jnp.dot(s, w1_ref[...], preferred_element_type=jnp.float32)
        h_ref[...] = jnp.maximum(h0_ref[...] + upd + be, 0.0)

    return pl.pallas_call(
        body, out_shape=jax.ShapeDtypeStruct((N_NODES, HIDDEN), jnp.float32)
    )(h0, s_p, be_p, w_h1)


def _tc_output(x, s_p, w_o1, w_o2, b_o):
    def body(x_ref, s_ref, w1_ref, w2_ref, b_ref, o_ref):
        a = s_ref[0, :N_NODES, :] + s_ref[1, :N_NODES, :]
        v = jnp.dot(x_ref[...], w1_ref[...], preferred_element_type=jnp.float32)
        v += jnp.dot(a, w2_ref[...], preferred_element_type=jnp.float32)
        o_ref[...] = jnp.maximum(v + b_ref[...], 0.0)

    return pl.pallas_call(
        body, out_shape=jax.ShapeDtypeStruct((N_NODES, HIDDEN), jnp.float32)
    )(x, s_p, w_o1, w_o2, b_o)


_RD_C = 2000
_RD_K = N_NODES // _RD_C


def _tc_readout_ffn(mol_ids3, h_atom, w_f1, b_f1, w_f2, b_f2):
    def body(m_ref, ha_ref, w1_ref, b1_ref, w2_ref, b2_ref, o_ref, acc, cnt):
        k = pl.program_id(0)

        @pl.when(k == 0)
        def _():
            acc[...] = jnp.zeros_like(acc)
            cnt[...] = jnp.zeros_like(cnt)

        m = m_ref[...].reshape(1, _RD_C)
        iota = lax.broadcasted_iota(jnp.int32, (N_MOLS, _RD_C), 0)
        oh = (iota == m).astype(jnp.float32)
        acc[...] += jnp.dot(oh, ha_ref[...], preferred_element_type=jnp.float32)
        cnt[...] += jnp.sum(oh, axis=1, keepdims=True)

        @pl.when(k == _RD_K - 1)
        def _():
            mol = acc[...] / jnp.maximum(cnt[...], 1.0)
            hdn = jnp.maximum(
                jnp.dot(mol, w1_ref[...], preferred_element_type=jnp.float32)
                + b1_ref[...],
                0.0,
            )
            o_ref[...] = (
                jnp.dot(hdn, w2_ref[...], preferred_element_type=jnp.float32)
                + b2_ref[...]
            )

    return pl.pallas_call(
        body,
        grid=(_RD_K,),
        in_specs=[
            pl.BlockSpec((1, 1, _RD_C), lambda k: (k, 0, 0)),
            pl.BlockSpec((_RD_C, HIDDEN), lambda k: (k, 0)),
            pl.BlockSpec((HIDDEN, HIDDEN), lambda k: (0, 0)),
            pl.BlockSpec((1, HIDDEN), lambda k: (0, 0)),
            pl.BlockSpec((HIDDEN, N_TASKS), lambda k: (0, 0)),
            pl.BlockSpec((1, N_TASKS), lambda k: (0, 0)),
        ],
        out_specs=pl.BlockSpec((N_MOLS, N_TASKS), lambda k: (0, 0)),
        out_shape=jax.ShapeDtypeStruct((N_MOLS, N_TASKS), jnp.float32),
        scratch_shapes=[
            pltpu.VMEM((N_MOLS, HIDDEN), jnp.float32),
            pltpu.VMEM((N_MOLS, 1), jnp.float32),
        ],
    )(mol_ids3, h_atom, w_f1, b_f1, w_f2, b_f2)


def kernel(x, edge_index, edge_attr, mol_ids, W_i, W_h, W_o, b_o, W_f1, b_f1, W_f2, b_f2):
    src = edge_index[0].astype(jnp.int32)
    dst = edge_index[1].astype(jnp.int32)
    src_s, dst_s = lax.sort((src, dst), num_keys=1)
    e = src.shape[0]
    per_sb = NW * SCW * CH
    nsc = -(-e // per_sb)
    nch = nsc * SCW
    e_pad = nsc * per_sb

    src_p = jnp.concatenate([src_s, jnp.zeros((e_pad - e,), jnp.int32)])
    dst_p = jnp.concatenate([dst_s, jnp.full((e_pad - e,), SENT, jnp.int32)])
    dst_po = jnp.concatenate([dst, jnp.full((e_pad - e,), SENT, jnp.int32)])
    pad_sb = jnp.zeros((NW, 1, SCW, CH), jnp.int32)
    idxi = jnp.concatenate(
        [jnp.arange(e_pad, dtype=jnp.int32).reshape(NW, nsc, SCW, CH), pad_sb], axis=1
    )
    dsti4 = jnp.concatenate([dst_po.reshape(NW, nsc, SCW, CH), pad_sb], axis=1)

    nsc1 = nsc
    nsc0 = 2 * nsc - nsc1
    nmax = max(nsc0, nsc1)

    def _part(arr, fill):
        cap0 = NS * nsc0 * SCW * CH
        p0 = arr[:cap0].reshape(NS, nsc0, SCW, CH)
        p1 = arr[cap0:].reshape(NS, nsc1, SCW, CH)
        f0 = jnp.full((NS, nmax + 1 - nsc0, SCW, CH), fill, jnp.int32)
        f1 = jnp.full((NS, nmax + 1 - nsc1, SCW, CH), fill, jnp.int32)
        return jnp.concatenate(
            [
                jnp.concatenate([p0, f0], axis=1),
                jnp.concatenate([p1, f1], axis=1),
            ],
            axis=0,
        )

    srci_rb = _part(src_p, 0)
    dsti_rb = _part(dst_p, SENT)

    ea = jnp.concatenate(
        [edge_attr, jnp.zeros((e_pad - e, D_EDGE), jnp.float32)], axis=0
    )
    zeros_h = jnp.zeros((NACC, HIDDEN), jnp.float32)

    seg_h = _seg_sum_gather(nsc0, nsc1)
    seg_lin = _seg_sum_gather(nsc, nsc)

    w_h1 = W_h[:HIDDEN]
    w_h2 = W_h[HIDDEN:]
    w_o1 = W_o[:D_FEAT]
    w_o2 = W_o[D_FEAT:]

    h0 = _tc_input(x, W_i)
    z = _tc_edge_proj(ea, w_h2)
    h0b, z = lax.optimization_barrier((h0, z))
    s_p = seg_h(h0b, h0b, srci_rb, dsti_rb, zeros_h)
    be_p = seg_lin(z, z, idxi, dsti4, zeros_h)
    h = _tc_update(h0, s_p, be_p, w_h1)
    for _ in range(DEPTH - 2):
        s_p = seg_h(h, h, srci_rb, dsti_rb, zeros_h)
        h = _tc_update(h0, s_p, be_p, w_h1)
    s_p = seg_h(h, h, srci_rb, dsti_rb, zeros_h)
    h_atom = _tc_output(x, s_p, w_o1, w_o2, b_o.reshape(1, HIDDEN))

    mol3 = mol_ids.astype(jnp.int32).reshape(_RD_K, 1, _RD_C)
    return _tc_readout_ffn(
        mol3, h_atom, W_f1, b_f1.reshape(1, HIDDEN), W_f2, b_f2.reshape(1, N_TASKS)
    )


# speedup vs baseline: 1.4074x; 1.4074x over previous
import functools

import jax
import jax.numpy as jnp
from jax import lax
from jax.experimental import pallas as pl
from jax.experimental.pallas import tpu as pltpu
from jax.experimental.pallas import tpu_sc as plsc

N_NODES = 10000
D_FEAT = 128
D_EDGE = 16
HIDDEN = 128
N_MOLS = 400
N_TASKS = 8
DEPTH = 3

NC = 2
NS = 16
NW = NC * NS
CH = 128
SCW = 16
NBUF = 2
NACC = 10112
RPS = NACC // NS
SENT = N_NODES

def _mesh():
    return plsc.VectorSubcoreMesh(core_axis_name="c", subcore_axis_name="s")


def _seg_sum_gather(nsc0, nsc1):
    nsc_max = max(nsc0, nsc1)

    @functools.partial(
        pl.kernel,
        out_type=jax.ShapeDtypeStruct((NC, NACC, HIDDEN), jnp.float32),
        mesh=_mesh(),
        scratch_types=[
            pltpu.VMEM((2, SCW, CH), jnp.int32),
            pltpu.VMEM((2, SCW, CH), jnp.int32),
            pltpu.VMEM((NBUF, CH, HIDDEN), jnp.float32),
            pltpu.VMEM_SHARED((NACC, HIDDEN), jnp.float32),
            [pltpu.SemaphoreType.DMA] * NBUF,
            pltpu.SemaphoreType.DMA,
            pltpu.SemaphoreType.DMA,
        ],
    )
    def k(table0, table1, srci, dsti, zeros, out, src_v, dst_v, bufs, acc,
          sems, sem_si, sem_di):
        cid = lax.axis_index("c")
        sid = lax.axis_index("s")
        w = cid * NS + sid
        pltpu.sync_copy(zeros.at[pl.ds(sid * RPS, RPS)], acc.at[pl.ds(sid * RPS, RPS)])
        pltpu.async_copy(srci.at[w, 0], src_v.at[0], sem_si)
        pltpu.async_copy(dsti.at[w, 0], dst_v.at[0], sem_di)
        plsc.subcore_barrier()

        def go(table, ns):
            @pl.loop(0, ns)
            def _(t):
                rt = lax.rem(t, 2)
                pltpu.make_async_copy(srci.at[w, 0], src_v.at[0], sem_si).wait()
                pltpu.make_async_copy(dsti.at[w, 0], dst_v.at[0], sem_di).wait()
                pltpu.async_copy(srci.at[w, t + 1], src_v.at[1 - rt], sem_si)
                pltpu.async_copy(dsti.at[w, t + 1], dst_v.at[1 - rt], sem_di)
                for b in range(NBUF):
                    pltpu.async_copy(table.at[src_v.at[rt, b]], bufs.at[b], sems[b])

                @pl.loop(0, SCW - NBUF, step=NBUF)
                def _(j):
                    for b in range(NBUF):
                        pltpu.make_async_copy(
                            table.at[src_v.at[rt, 0]], bufs.at[b], sems[b]
                        ).wait()
                        pltpu.sync_copy(
                            bufs.at[b], acc.at[dst_v.at[rt, j + b]], add=True
                        )
                        pltpu.async_copy(
                            table.at[src_v.at[rt, j + b + NBUF]], bufs.at[b], sems[b]
                        )

                for b in range(NBUF):
                    pltpu.make_async_copy(
                        table.at[src_v.at[rt, 0]], bufs.at[b], sems[b]
                    ).wait()
                    pltpu.sync_copy(
                        bufs.at[b], acc.at[dst_v.at[rt, SCW - NBUF + b]], add=True
                    )

        @pl.when(cid == 0)
        def _():
            go(table0, nsc0)

        @pl.when(cid == 1)
        def _():
            go(table1, nsc1)

        pltpu.make_async_copy(srci.at[w, 0], src_v.at[0], sem_si).wait()
        pltpu.make_async_copy(dsti.at[w, 0], dst_v.at[0], sem_di).wait()
        plsc.subcore_barrier()
        pltpu.sync_copy(acc.at[pl.ds(sid * RPS, RPS)], out.at[cid].at[pl.ds(sid * RPS, RPS)])

    return k


def _tc_input(x, w_i):
    def body(x_ref, w_ref, o_ref):
        o_ref[...] = jnp.maximum(
            jnp.dot(x_ref[...], w_ref[...], preferred_element_type=jnp.float32), 0.0
        )

    return pl.pallas_call(
        body, out_shape=jax.ShapeDtypeStruct((N_NODES, HIDDEN), jnp.float32)
    )(x, w_i)


_EP_C = 8192


def _tc_edge_proj(ea, w_h2):
    n = ea.shape[0]
    grid = n // _EP_C

    def body(ea_ref, w_ref, z_ref):
        z_ref[...] = jnp.dot(ea_ref[...], w_ref[...], preferred_element_type=jnp.float32)

    return pl.pallas_call(
        body,
        grid=(grid,),
        in_specs=[
            pl.BlockSpec((_EP_C, D_EDGE), lambda k: (k, 0)),
            pl.BlockSpec((D_EDGE, HIDDEN), lambda k: (0, 0)),
        ],
        out_specs=pl.BlockSpec((_EP_C, HIDDEN), lambda k: (k, 0)),
        out_shape=jax.ShapeDtypeStruct((n, HIDDEN), jnp.float32),
    )(ea, w_h2)


def _tc_update(h0, s_p, be_p, w_h1):
    def body(h0_ref, s_ref, be_ref, w1_ref, h_ref):
        s = s_ref[0, :N_NODES, :] + s_ref[1, :N_NODES, :]
        be = be_ref[0, :N_NODES, :] + be_ref[1, :N_NODES, :]
        upd = jnp.dot(s, w1_ref[...], preferred_element_type=jnp.float32)
        h_ref[...] = jnp.maximum(h0_ref[...] + upd + be, 0.0)

    return pl.pallas_call(
        body, out_shape=jax.ShapeDtypeStruct((N_NODES, HIDDEN), jnp.float32)
    )(h0, s_p, be_p, w_h1)


def _tc_output(x, s_p, w_o1, w_o2, b_o):
    def body(x_ref, s_ref, w1_ref, w2_ref, b_ref, o_ref):
        a = s_ref[0, :N_NODES, :] + s_ref[1, :N_NODES, :]
        v = jnp.dot(x_ref[...], w1_ref[...], preferred_element_type=jnp.float32)
        v += jnp.dot(a, w2_ref[...], preferred_element_type=jnp.float32)
        o_ref[...] = jnp.maximum(v + b_ref[...], 0.0)

    return pl.pallas_call(
        body, out_shape=jax.ShapeDtypeStruct((N_NODES, HIDDEN), jnp.float32)
    )(x, s_p, w_o1, w_o2, b_o)


_RD_C = 2000
_RD_K = N_NODES // _RD_C


def _tc_readout_ffn(mol_ids3, h_atom, w_f1, b_f1, w_f2, b_f2):
    def body(m_ref, ha_ref, w1_ref, b1_ref, w2_ref, b2_ref, o_ref, acc, cnt):
        k = pl.program_id(0)

        @pl.when(k == 0)
        def _():
            acc[...] = jnp.zeros_like(acc)
            cnt[...] = jnp.zeros_like(cnt)

        m = m_ref[...].reshape(1, _RD_C)
        iota = lax.broadcasted_iota(jnp.int32, (N_MOLS, _RD_C), 0)
        oh = (iota == m).astype(jnp.float32)
        acc[...] += jnp.dot(oh, ha_ref[...], preferred_element_type=jnp.float32)
        cnt[...] += jnp.sum(oh, axis=1, keepdims=True)

        @pl.when(k == _RD_K - 1)
        def _():
            mol = acc[...] / jnp.maximum(cnt[...], 1.0)
            hdn = jnp.maximum(
                jnp.dot(mol, w1_ref[...], preferred_element_type=jnp.float32)
                + b1_ref[...],
                0.0,
            )
            o_ref[...] = (
                jnp.dot(hdn, w2_ref[...], preferred_element_type=jnp.float32)
                + b2_ref[...]
            )

    return pl.pallas_call(
        body,
        grid=(_RD_K,),
        in_specs=[
            pl.BlockSpec((1, 1, _RD_C), lambda k: (k, 0, 0)),
            pl.BlockSpec((_RD_C, HIDDEN), lambda k: (k, 0)),
            pl.BlockSpec((HIDDEN, HIDDEN), lambda k: (0, 0)),
            pl.BlockSpec((1, HIDDEN), lambda k: (0, 0)),
            pl.BlockSpec((HIDDEN, N_TASKS), lambda k: (0, 0)),
            pl.BlockSpec((1, N_TASKS), lambda k: (0, 0)),
        ],
        out_specs=pl.BlockSpec((N_MOLS, N_TASKS), lambda k: (0, 0)),
        out_shape=jax.ShapeDtypeStruct((N_MOLS, N_TASKS), jnp.float32),
        scratch_shapes=[
            pltpu.VMEM((N_MOLS, HIDDEN), jnp.float32),
            pltpu.VMEM((N_MOLS, 1), jnp.float32),
        ],
    )(mol_ids3, h_atom, w_f1, b_f1, w_f2, b_f2)


def kernel(x, edge_index, edge_attr, mol_ids, W_i, W_h, W_o, b_o, W_f1, b_f1, W_f2, b_f2):
    src = edge_index[0].astype(jnp.int32)
    dst = edge_index[1].astype(jnp.int32)
    e = src.shape[0]
    per_sb = NW * SCW * CH
    nsc = -(-e // per_sb)
    nch = nsc * SCW
    e_pad = nsc * per_sb

    src_p = jnp.concatenate([src, jnp.zeros((e_pad - e,), jnp.int32)])
    dst_p = jnp.concatenate([dst, jnp.full((e_pad - e,), SENT, jnp.int32)])
    pad_sb = jnp.zeros((NW, 1, SCW, CH), jnp.int32)
    idxi = jnp.concatenate(
        [jnp.arange(e_pad, dtype=jnp.int32).reshape(NW, nsc, SCW, CH), pad_sb], axis=1
    )
    dsti4 = jnp.concatenate([dst_p.reshape(NW, nsc, SCW, CH), pad_sb], axis=1)

    nsc1 = max(2 * nsc // 10, 1)
    nsc0 = 2 * nsc - nsc1
    nmax = max(nsc0, nsc1)

    def _part(arr, fill):
        cap0 = NS * nsc0 * SCW * CH
        p0 = arr[:cap0].reshape(NS, nsc0, SCW, CH)
        p1 = arr[cap0:].reshape(NS, nsc1, SCW, CH)
        f0 = jnp.full((NS, nmax + 1 - nsc0, SCW, CH), fill, jnp.int32)
        f1 = jnp.full((NS, nmax + 1 - nsc1, SCW, CH), fill, jnp.int32)
        return jnp.concatenate(
            [
                jnp.concatenate([p0, f0], axis=1),
                jnp.concatenate([p1, f1], axis=1),
            ],
            axis=0,
        )

    srci_rb = _part(src_p, 0)
    dsti_rb = _part(dst_p, SENT)

    ea = jnp.concatenate(
        [edge_attr, jnp.zeros((e_pad - e, D_EDGE), jnp.float32)], axis=0
    )
    zeros_h = jnp.zeros((NACC, HIDDEN), jnp.float32)

    seg_h = _seg_sum_gather(nsc0, nsc1)
    seg_lin = _seg_sum_gather(nsc, nsc)

    w_h1 = W_h[:HIDDEN]
    w_h2 = W_h[HIDDEN:]
    w_o1 = W_o[:D_FEAT]
    w_o2 = W_o[D_FEAT:]

    h0 = _tc_input(x, W_i)
    z = _tc_edge_proj(ea, w_h2)
    h0b, z = lax.optimization_barrier((h0, z))
    hs = jnp.stack([h0b, h0b])
    s_p = seg_h(hs[0], hs[1], srci_rb, dsti_rb, zeros_h)
    be_p = seg_lin(z, z, idxi, dsti4, zeros_h)
    h = _tc_update(h0, s_p, be_p, w_h1)
    for _ in range(DEPTH - 2):
        hs = jnp.stack([h, h])
        s_p = seg_h(hs[0], hs[1], srci_rb, dsti_rb, zeros_h)
        h = _tc_update(h0, s_p, be_p, w_h1)
    hs = jnp.stack([h, h])
    s_p = seg_h(hs[0], hs[1], srci_rb, dsti_rb, zeros_h)
    h_atom = _tc_output(x, s_p, w_o1, w_o2, b_o.reshape(1, HIDDEN))

    mol3 = mol_ids.astype(jnp.int32).reshape(_RD_K, 1, _RD_C)
    return _tc_readout_ffn(
        mol3, h_atom, W_f1, b_f1.reshape(1, HIDDEN), W_f2, b_f2.reshape(1, N_TASKS)
    )
